# baseline (speedup 1.0000x reference)
import functools

import jax
import jax.numpy as jnp
from jax import lax
from jax.experimental import pallas as pl
from jax.experimental.pallas import tpu as pltpu
from jax.experimental.pallas import tpu_sc as plsc

_B, _K, _C, _W, _H = 256, 133, 256, 768, 1024
_KTOP = 10
_TB = 8
_NB = _B // _TB
_NC, _NS, _L = 2, 16, 16
_NW = _NC * _NS
_KCH = 9


def _scores_body(hx_ref, hy_ref, out_ref):
    cols = []
    for tb in range(_TB):
        hx = hx_ref[tb]
        hy = hy_ref[tb]
        sx = jnp.sum(jnp.exp(hx - jnp.max(hx, axis=-1, keepdims=True)),
                     axis=-1, keepdims=True)
        sy = jnp.sum(jnp.exp(hy - jnp.max(hy, axis=-1, keepdims=True)),
                     axis=-1, keepdims=True)
        cols.append(1.0 - 0.5 * (1.0 / sx + 1.0 / sy))
    out_ref[0] = jnp.concatenate(cols, axis=1)


def _sc_body(scores_hbm, f_hbm, canon_hbm, outf_hbm, mask_hbm,
             slab_v, fbuf_v, rows_v, idx_v, mbuf_v, sem):
    wid = lax.axis_index("s") * _NC + lax.axis_index("c")
    lane = lax.iota(jnp.int32, _L)

    pltpu.sync_copy(scores_hbm.at[wid], slab_v)

    for j in range(_TB):
        cur = []
        col = jnp.full((_L,), j, jnp.int32)
        for i in range(_KCH):
            ridx = jnp.minimum(lane + 16 * i, _K - 1)
            v = plsc.load_gather(slab_v, [ridx, col])
            v = jnp.where(lane + 16 * i < _K, v, -1.0)
            cur.append(v)

        msel = [jnp.zeros((_L,), jnp.bool_) for _ in range(_KCH)]
        idxvec = jnp.zeros((_L,), jnp.int32)
        idx = jnp.int32(0)
        for t in range(_KTOP):
            mvec = cur[0]
            for i in range(1, _KCH):
                mvec = jnp.maximum(mvec, cur[i])
            m = jnp.max(mvec)
            cand = jnp.full((_L,), 10000, jnp.int32)
            for i in range(_KCH):
                cand = jnp.minimum(cand,
                                   jnp.where(cur[i] == m, lane + 16 * i, 10000))
            idx = jnp.min(cand)
            idxvec = jnp.where(lane == t, idx, idxvec)
            for i in range(_KCH):
                hit = (lane + 16 * i) == idx
                msel[i] = msel[i] | hit
                cur[i] = jnp.where(hit, -2.0, cur[i])
        idxvec = jnp.where(lane < _KTOP, idxvec, idx)
        idx_v[...] = idxvec

        for i in range(_KCH):
            mbuf_v[pl.ds(16 * i, 16)] = msel[i].astype(jnp.int32)
        pltpu.sync_copy(mbuf_v, mask_hbm.at[wid * _TB + j])

        pltpu.sync_copy(f_hbm.at[wid * _TB + j], fbuf_v)
        pltpu.async_copy(canon_hbm.at[idx_v], rows_v, sem).wait()
        pltpu.sync_copy(fbuf_v, outf_hbm.at[wid * _TB + j])
        pltpu.async_copy(rows_v, outf_hbm.at[wid * _TB + j].at[idx_v],
                         sem).wait()


_sc_call = functools.partial(
    pl.kernel,
    out_type=[
        jax.ShapeDtypeStruct((_B, _K, _C), jnp.float32),
        jax.ShapeDtypeStruct((_B, _KCH * _L), jnp.int32),
    ],
    mesh=plsc.VectorSubcoreMesh(core_axis_name="c", subcore_axis_name="s"),
    compiler_params=pltpu.CompilerParams(needs_layout_passes=False),
    scratch_types=[
        pltpu.VMEM((_K, _TB), jnp.float32),
        pltpu.VMEM((_K, _C), jnp.float32),
        pltpu.VMEM((_L, _C), jnp.float32),
        pltpu.VMEM((_L,), jnp.int32),
        pltpu.VMEM((_KCH * _L,), jnp.int32),
        pltpu.SemaphoreType.DMA,
    ],
)


def kernel(f_kpts, h_initial_x, h_initial_y, canonical_table):
    scores_t = pl.pallas_call(
        _scores_body,
        grid=(_NB,),
        in_specs=[
            pl.BlockSpec((_TB, _K, _W), lambda i: (i, 0, 0)),
            pl.BlockSpec((_TB, _K, _H), lambda i: (i, 0, 0)),
        ],
        out_specs=pl.BlockSpec((1, _K, _TB), lambda i: (i, 0, 0)),
        out_shape=jax.ShapeDtypeStruct((_NB, _K, _TB), jnp.float32),
        compiler_params=pltpu.CompilerParams(
            dimension_semantics=("parallel",),
        ),
    )(h_initial_x, h_initial_y)

    out_f, mask_rows = _sc_call(_sc_body)(scores_t, f_kpts, canonical_table)
    return out_f, (mask_rows[:, :_K] != 0)
